# Initial kernel scaffold; baseline (speedup 1.0000x reference)
import functools

import numpy as np
import jax
import jax.numpy as jnp
from jax import lax
from jax.experimental import pallas as pl
from jax.experimental.pallas import tpu as pltpu
from jax.experimental.pallas import tpu_sc as plsc

N = 10000
E = 320000
HID = 128
OUT = 128
H = 8
DK = 16

NC = 2
NS = 16
NW = NC * NS
B = 80
CHUNKS_PER_W = E // (NW * B)
ROWS_PER_SUB = N // NS
ZROWS = 125

ROW_BLK = 1000


def _build_perm() -> np.ndarray:
    idx = np.zeros(HID, dtype=np.int32)
    for h in range(H):
        for d in range(DK):
            p = (d // 2) * 16 + (h if d % 2 == 0 else 15 - h)
            idx[p] = h * DK + d
    return idx


_PERM = _build_perm()

_TILE16 = np.zeros((16, HID), dtype=np.float32)
for _c in range(8):
    for _l in range(16):
        _TILE16[_l, _c * 16 + _l] = 1.0


def _proj_body(x_ref, wq_ref, bq_ref, wk_ref, wv_ref, q_ref, k_ref, v_ref):
    x = x_ref[...]
    dn = (((1,), (1,)), ((), ()))
    q_ref[...] = lax.dot_general(x, wq_ref[...], dn,
                                 preferred_element_type=jnp.float32) + bq_ref[...]
    k_ref[...] = lax.dot_general(x, wk_ref[...], dn,
                                 preferred_element_type=jnp.float32)
    v_ref[...] = lax.dot_general(x, wv_ref[...], dn,
                                 preferred_element_type=jnp.float32)


def _project(x, wq_p, bq_p, wk_p, wv_p):
    f32 = jnp.float32
    full = lambda s: pl.BlockSpec(s, lambda i: (0, 0))
    row = pl.BlockSpec((ROW_BLK, HID), lambda i: (i, 0))
    return pl.pallas_call(
        _proj_body,
        grid=(N // ROW_BLK,),
        in_specs=[row, full((HID, HID)), full((1, HID)), full((HID, HID)),
                  full((HID, HID))],
        out_specs=[row, row, row],
        out_shape=[jax.ShapeDtypeStruct((N, HID), f32)] * 3,
    )(x, wq_p, bq_p, wk_p, wv_p)


def _sc_edges(k, q, v, edge_index):
    f32 = jnp.float32
    mesh = plsc.VectorSubcoreMesh(core_axis_name="c", subcore_axis_name="s")

    @functools.partial(
        pl.kernel,
        out_type=[jax.ShapeDtypeStruct((NC, N, HID), f32),
                  jax.ShapeDtypeStruct((NC, N, 16), f32)],
        mesh=mesh,
        scratch_types=[
            pltpu.VMEM((B,), jnp.int32),
            pltpu.VMEM((B,), jnp.int32),
            pltpu.VMEM((B, HID), f32),
            pltpu.VMEM((B, HID), f32),
            pltpu.VMEM((B, HID), f32),
            pltpu.VMEM((B, HID), f32),
            pltpu.VMEM((B, 16), f32),
            pltpu.VMEM((ZROWS, HID), f32),
            pltpu.VMEM((ZROWS, 16), f32),
            pltpu.VMEM_SHARED((N, HID), f32),
            pltpu.VMEM_SHARED((N, 16), f32),
            pltpu.SemaphoreType.DMA,
            pltpu.SemaphoreType.DMA,
            pltpu.SemaphoreType.DMA,
        ],
    )
    def sc_kernel(k_hbm, q_hbm, v_hbm, ei_hbm, wv_hbm, z_hbm,
                  src_v, dst_v, k_rows, q_rows, v_rows, msg_v, sc_v,
                  zwv, zz, wv_sh, z_sh, sem0, sem1, sem2):
        cid = lax.axis_index("c")
        sid = lax.axis_index("s")
        wid = sid * NC + cid

        zero16 = jnp.zeros((16,), f32)

        @pl.loop(0, ZROWS)
        def _zero_fill(r):
            for c in range(8):
                zwv[r, pl.ds(c * 16, 16)] = zero16
            zz[r, :] = zero16

        @pl.loop(0, ROWS_PER_SUB // ZROWS)
        def _zero_shared(j):
            base = sid * ROWS_PER_SUB + j * ZROWS
            pltpu.sync_copy(zwv, wv_sh.at[pl.ds(base, ZROWS)])
            pltpu.sync_copy(zz, z_sh.at[pl.ds(base, ZROWS)])

        plsc.subcore_barrier()

        @pl.loop(0, CHUNKS_PER_W)
        def _chunk(j):
            off = wid * (CHUNKS_PER_W * B) + j * B
            pltpu.sync_copy(ei_hbm.at[0, pl.ds(off, B)], src_v)
            pltpu.sync_copy(ei_hbm.at[1, pl.ds(off, B)], dst_v)
            ck = pltpu.async_copy(k_hbm.at[src_v], k_rows, sem0)
            cq = pltpu.async_copy(q_hbm.at[dst_v], q_rows, sem1)
            cv = pltpu.async_copy(v_hbm.at[src_v], v_rows, sem2)
            ck.wait()
            cq.wait()
            cv.wait()

            @pl.loop(0, B)
            def _edge(i):
                acc = k_rows[i, pl.ds(0, 16)] * q_rows[i, pl.ds(0, 16)]
                for c in range(1, 8):
                    acc += k_rows[i, pl.ds(c * 16, 16)] * q_rows[i, pl.ds(c * 16, 16)]
                t = (acc + lax.rev(acc, (0,))) * 0.25
                t = jnp.minimum(jnp.maximum(t, -5.0), 5.0)
                s = jnp.exp(t)
                sc_v[i, :] = s
                for c in range(8):
                    msg_v[i, pl.ds(c * 16, 16)] = v_rows[i, pl.ds(c * 16, 16)] * s

            pltpu.sync_copy(msg_v, wv_sh.at[dst_v], add=True)
            pltpu.sync_copy(sc_v, z_sh.at[dst_v], add=True)

        plsc.subcore_barrier()

        base = sid * ROWS_PER_SUB
        pltpu.sync_copy(wv_sh.at[pl.ds(base, ROWS_PER_SUB)],
                        wv_hbm.at[cid, pl.ds(base, ROWS_PER_SUB)])
        pltpu.sync_copy(z_sh.at[pl.ds(base, ROWS_PER_SUB)],
                        z_hbm.at[cid, pl.ds(base, ROWS_PER_SUB)])

    return sc_kernel(k, q, v, edge_index)


def _out_body(wv0_ref, wv1_ref, z0_ref, z1_ref, t_ref, wo_ref, bo_ref, out_ref):
    wv = wv0_ref[...] + wv1_ref[...]
    z = z0_ref[...] + z1_ref[...]
    den = lax.dot_general(z, t_ref[...], (((1,), (0,)), ((), ())),
                          preferred_element_type=jnp.float32)
    o = wv / den
    out_ref[...] = lax.dot_general(o, wo_ref[...], (((1,), (1,)), ((), ())),
                                   preferred_element_type=jnp.float32) + bo_ref[...]


def _finish(wv0, wv1, z0, z1, tile16, wo_p, bo2):
    f32 = jnp.float32
    row = pl.BlockSpec((ROW_BLK, HID), lambda i: (i, 0))
    zrow = pl.BlockSpec((ROW_BLK, 16), lambda i: (i, 0))
    full = lambda s: pl.BlockSpec(s, lambda i: (0, 0))
    return pl.pallas_call(
        _out_body,
        grid=(N // ROW_BLK,),
        in_specs=[row, row, zrow, zrow, full((16, HID)), full((OUT, HID)),
                  full((1, OUT))],
        out_specs=pl.BlockSpec((ROW_BLK, OUT), lambda i: (i, 0)),
        out_shape=jax.ShapeDtypeStruct((N, OUT), f32),
    )(wv0, wv1, z0, z1, tile16, wo_p, bo2)


def kernel(inputs, edge_index, Wq, bq, Wk, Wv, Wo, bo):
    perm = jnp.asarray(_PERM)
    wq_p = Wq[perm]
    bq_p = bq[perm].reshape(1, HID)
    wk_p = Wk[perm]
    wv_p = Wv[perm]
    wo_p = Wo[:, perm]
    tile16 = jnp.asarray(_TILE16)

    q, k, v = _project(inputs, wq_p, bq_p, wk_p, wv_p)
    wv_parts, z_parts = _sc_edges(k, q, v, edge_index)
    out = _finish(wv_parts[0], wv_parts[1], z_parts[0], z_parts[1],
                  tile16, wo_p, bo.reshape(1, OUT))
    return out


# speedup vs baseline: 45.9675x; 45.9675x over previous
import functools

import numpy as np
import jax
import jax.numpy as jnp
from jax import lax
from jax.experimental import pallas as pl
from jax.experimental.pallas import tpu as pltpu
from jax.experimental.pallas import tpu_sc as plsc

N = 10000
E = 320000
HID = 128
OUT = 128
H = 8
DK = 16

NC = 2
NS = 16
NW = NC * NS
B = 64
G = B // 16
NCHUNKS = E // B
CHUNK_ITERS = -(-NCHUNKS // NW)
NPAD = 10240
ROWS_PER_SUB = NPAD // NS
ZP_ROWS = NPAD // 8
ZP_PER_SUB = ZP_ROWS // NS

ROW_BLK = 1000


def _build_perm() -> np.ndarray:
    idx = np.zeros(HID, dtype=np.int32)
    for h in range(H):
        for d in range(DK):
            p = (d // 2) * 16 + (h if d % 2 == 0 else 15 - h)
            idx[p] = h * DK + d
    return idx


_PERM = _build_perm()

_TILE16 = np.zeros((16, HID), dtype=np.float32)
for _c in range(8):
    for _l in range(16):
        _TILE16[_l, _c * 16 + _l] = 1.0


def _proj_body(x_ref, wq_ref, bq_ref, wk_ref, wv_ref, q_ref, k_ref, v_ref):
    x = x_ref[...]
    dn = (((1,), (1,)), ((), ()))
    q_ref[...] = lax.dot_general(x, wq_ref[...], dn,
                                 preferred_element_type=jnp.float32) + bq_ref[...]
    k_ref[...] = lax.dot_general(x, wk_ref[...], dn,
                                 preferred_element_type=jnp.float32)
    v_ref[...] = lax.dot_general(x, wv_ref[...], dn,
                                 preferred_element_type=jnp.float32)


def _project(x, wq_p, bq_p, wk_p, wv_p):
    f32 = jnp.float32
    full = lambda s: pl.BlockSpec(s, lambda i: (0, 0))
    row = pl.BlockSpec((ROW_BLK, HID), lambda i: (i, 0))
    return pl.pallas_call(
        _proj_body,
        grid=(N // ROW_BLK,),
        in_specs=[row, full((HID, HID)), full((1, HID)), full((HID, HID)),
                  full((HID, HID))],
        out_specs=[row, row, row],
        out_shape=[jax.ShapeDtypeStruct((N, HID), f32)] * 3,
    )(x, wq_p, bq_p, wk_p, wv_p)


def _sc_edges(k, q, v, edge_index):
    f32 = jnp.float32
    mesh = plsc.VectorSubcoreMesh(core_axis_name="c", subcore_axis_name="s")

    @functools.partial(
        pl.kernel,
        out_type=[jax.ShapeDtypeStruct((NC, NPAD, HID), f32),
                  jax.ShapeDtypeStruct((NC, ZP_ROWS, HID), f32)],
        mesh=mesh,
        scratch_types=[
            pltpu.VMEM((B,), jnp.int32),
            pltpu.VMEM((B,), jnp.int32),
            pltpu.VMEM((B,), jnp.int32),
            pltpu.VMEM((B, HID), f32),
            pltpu.VMEM((B, HID), f32),
            pltpu.VMEM((B, HID), f32),
            pltpu.VMEM((B, 16), f32),
            pltpu.VMEM_SHARED((NPAD, HID), f32),
            pltpu.VMEM_SHARED((ZP_ROWS, HID), f32),
            pltpu.SemaphoreType.DMA,
            pltpu.SemaphoreType.DMA,
            pltpu.SemaphoreType.DMA,
        ],
    )
    def sc_kernel(k_hbm, q_hbm, v_hbm, src_hbm, dst_hbm, wv_hbm, z_hbm,
                  src_v, dst_v, zrid_v, k_rows, q_rows, v_rows, sc_v,
                  wv_sh, zp_sh, sem0, sem1, sem2):
        cid = lax.axis_index("c")
        sid = lax.axis_index("s")
        wid = sid * NC + cid

        zero16 = jnp.zeros((16,), f32)

        @pl.loop(0, B)
        def _zero_fill(r):
            for c in range(8):
                k_rows[r, pl.ds(c * 16, 16)] = zero16

        @pl.loop(0, ROWS_PER_SUB // B)
        def _zero_wv(j):
            base = sid * ROWS_PER_SUB + j * B
            pltpu.sync_copy(k_rows, wv_sh.at[pl.ds(base, B)])

        @pl.loop(0, 2)
        def _zero_zp(j):
            pltpu.sync_copy(
                k_rows.at[pl.ds(0, ZP_PER_SUB // 2)],
                zp_sh.at[pl.ds(sid * ZP_PER_SUB + j * (ZP_PER_SUB // 2),
                               ZP_PER_SUB // 2)])

        plsc.subcore_barrier()

        @pl.loop(0, CHUNK_ITERS)
        def _chunk(j):
          chunk = wid + j * NW

          @pl.when(chunk < NCHUNKS)
          def _do_chunk():
            off = chunk * B
            pltpu.sync_copy(src_hbm.at[pl.ds(off, B)], src_v)
            pltpu.sync_copy(dst_hbm.at[pl.ds(off, B)], dst_v)
            ck = pltpu.async_copy(k_hbm.at[src_v], k_rows, sem0)
            cq = pltpu.async_copy(q_hbm.at[dst_v], q_rows, sem1)
            cv = pltpu.async_copy(v_hbm.at[src_v], v_rows, sem2)
            ck.wait()
            cq.wait()
            cv.wait()

            @pl.loop(0, G)
            def _edge_group(g):
                for t in range(16):
                    i = g * 16 + t
                    acc = k_rows[i, pl.ds(0, 16)] * q_rows[i, pl.ds(0, 16)]
                    for c in range(1, 8):
                        acc += (k_rows[i, pl.ds(c * 16, 16)]
                                * q_rows[i, pl.ds(c * 16, 16)])
                    ts = (acc + lax.rev(acc, (0,))) * 0.25
                    ts = jnp.minimum(jnp.maximum(ts, -5.0), 5.0)
                    s = jnp.exp(ts)
                    sc_v[i, :] = s
                    for c in range(8):
                        v_rows[i, pl.ds(c * 16, 16)] = (
                            v_rows[i, pl.ds(c * 16, 16)] * s)

            pltpu.sync_copy(v_rows, wv_sh.at[dst_v], add=True)

            @pl.loop(0, G)
            def _z_group(g):
                d16 = dst_v[pl.ds(g * 16, 16)]
                zrid_v[pl.ds(g * 16, 16)] = lax.shift_right_logical(d16, 3)
                for t in range(16):
                    i = g * 16 + t
                    for c in range(8):
                        k_rows[i, pl.ds(c * 16, 16)] = zero16
                    m = lax.rem(d16[t], 8)
                    k_rows[i, pl.ds(m * 16, 16)] = sc_v[i, :]

            pltpu.sync_copy(k_rows, zp_sh.at[zrid_v], add=True)

        plsc.subcore_barrier()

        base = sid * ROWS_PER_SUB
        pltpu.sync_copy(wv_sh.at[pl.ds(base, ROWS_PER_SUB)],
                        wv_hbm.at[cid, pl.ds(base, ROWS_PER_SUB)])
        zbase = sid * ZP_PER_SUB
        pltpu.sync_copy(zp_sh.at[pl.ds(zbase, ZP_PER_SUB)],
                        z_hbm.at[cid, pl.ds(zbase, ZP_PER_SUB)])

    return sc_kernel(k, q, v, edge_index[0], edge_index[1])


def _out_body(wv0_ref, wv1_ref, z0_ref, z1_ref, t_ref, wo_ref, bo_ref, out_ref):
    wv = wv0_ref[...] + wv1_ref[...]
    z = z0_ref[...] + z1_ref[...]
    den = lax.dot_general(z, t_ref[...], (((1,), (0,)), ((), ())),
                          preferred_element_type=jnp.float32)
    o = wv / den
    out_ref[...] = lax.dot_general(o, wo_ref[...], (((1,), (1,)), ((), ())),
                                   preferred_element_type=jnp.float32) + bo_ref[...]


def _finish(wv0, wv1, z0, z1, tile16, wo_p, bo2):
    f32 = jnp.float32
    row = pl.BlockSpec((ROW_BLK, HID), lambda i: (i, 0))
    zrow = pl.BlockSpec((ROW_BLK, 16), lambda i: (i, 0))
    full = lambda s: pl.BlockSpec(s, lambda i: (0, 0))
    return pl.pallas_call(
        _out_body,
        grid=(N // ROW_BLK,),
        in_specs=[row, row, zrow, zrow, full((16, HID)), full((OUT, HID)),
                  full((1, OUT))],
        out_specs=pl.BlockSpec((ROW_BLK, OUT), lambda i: (i, 0)),
        out_shape=jax.ShapeDtypeStruct((N, OUT), f32),
    )(wv0, wv1, z0, z1, tile16, wo_p, bo2)


def kernel(inputs, edge_index, Wq, bq, Wk, Wv, Wo, bo):
    perm = jnp.asarray(_PERM)
    wq_p = Wq[perm]
    bq_p = bq[perm].reshape(1, HID)
    wk_p = Wk[perm]
    wv_p = Wv[perm]
    wo_p = Wo[:, perm]
    tile16 = jnp.asarray(_TILE16)

    q, k, v = _project(inputs, wq_p, bq_p, wk_p, wv_p)
    wv_parts, z_parts = _sc_edges(k, q, v, edge_index)
    z_flat = z_parts.reshape(NC, NPAD, 16)
    out = _finish(wv_parts[0], wv_parts[1], z_flat[0], z_flat[1],
                  tile16, wo_p, bo.reshape(1, OUT))
    return out
